# baseline (speedup 1.0000x reference)
import jax
import jax.numpy as jnp
from jax import lax
from jax.experimental import pallas as pl
from jax.experimental.pallas import tpu as pltpu
from jax.experimental.pallas import tpu_sc as plsc

NUM_ENTS = 100000
H_DIM = 128
BATCH = 16384
ALPHA = 0.2

_NC = 2
_NS = 16
_NW = _NC * _NS
_RPW = BATCH // _NW
_CHUNK = 128
_NCHUNK = _RPW // _CHUNK
_NROW = BATCH // _CHUNK

_WIN_CHUNK = 2048
_UNROLL = 8

_DENSE_BLK = 4096
_COPY_BLK = 10000


def _dense_body(scal_ref, h_ref, ctx_ref, slow_ref, fast_ref,
                hout_ref, fastnew_ref, slowupd_ref):
    sqrt_c = jnp.sqrt(scal_ref[0])
    thr = jnp.maximum(scal_ref[1], 1e-6)
    scale = jnp.maximum(scal_ref[2], 0.1)

    h = h_ref[...]
    slow = slow_ref[...]

    n1 = jnp.maximum(jnp.sqrt(jnp.sum(h * h, axis=-1, keepdims=True)), 1e-10)
    x = jnp.clip(sqrt_c * n1, -1.0 + 1e-7, 1.0 - 1e-7)
    artanh = 0.5 * jnp.log((1.0 + x) / (1.0 - x))
    t = artanh * h / (sqrt_c * n1) + slow
    t = jnp.clip(t, -10.0, 10.0)

    n2 = jnp.maximum(jnp.sqrt(jnp.sum(t * t, axis=-1, keepdims=True)), 1e-10)
    arg = jnp.clip(sqrt_c * n2, -15.0, 15.0)
    e = jnp.tanh(arg) * t / (sqrt_c * n2)

    n3 = jnp.maximum(jnp.sqrt(jnp.sum(e * e, axis=-1, keepdims=True)), 1e-10)
    max_norm = (1.0 - 1e-5) / sqrt_c
    hout_ref[...] = e * jnp.minimum(1.0, max_norm / n3)

    fast_new = (1.0 - ALPHA) * fast_ref[...] + ALPHA * ctx_ref[...]
    fastnew_ref[...] = fast_new
    delta = fast_new - slow
    dn = jnp.sqrt(jnp.sum(delta * delta, axis=-1, keepdims=True))
    gate = jax.nn.sigmoid(scale * (dn - thr))
    slowupd_ref[...] = slow + gate * delta


def _dense_compute(scal, h_hyp, context_tangent, slow_rows, fast_rows):
    grid = (BATCH // _DENSE_BLK,)
    row_spec = pl.BlockSpec((_DENSE_BLK, H_DIM), lambda i: (i, 0))
    return pl.pallas_call(
        _dense_body,
        grid=grid,
        in_specs=[
            pl.BlockSpec(memory_space=pltpu.SMEM),
            row_spec, row_spec, row_spec, row_spec,
        ],
        out_specs=[row_spec, row_spec, row_spec],
        out_shape=[
            jax.ShapeDtypeStruct((BATCH, H_DIM), jnp.float32),
            jax.ShapeDtypeStruct((BATCH, H_DIM), jnp.float32),
            jax.ShapeDtypeStruct((BATCH, H_DIM), jnp.float32),
        ],
    )(scal, h_hyp, context_tangent, slow_rows, fast_rows)


def _copy_body(src_ref, dst_ref):
    dst_ref[...] = src_ref[...]


def _copy_buffer(buf):
    blk = pl.BlockSpec((_COPY_BLK, H_DIM), lambda i: (i, 0))
    return pl.pallas_call(
        _copy_body,
        grid=(NUM_ENTS // _COPY_BLK,),
        in_specs=[blk],
        out_specs=blk,
        out_shape=jax.ShapeDtypeStruct((NUM_ENTS, H_DIM), jnp.float32),
    )(buf)


_TAIL = 57344
_HEAD = NUM_ENTS - _TAIL
_TPW = _TAIL // _NW
_HEAD_BLK = 1376


def _head_copy_body(_base_ref, src_ref, dst_ref):
    dst_ref[...] = src_ref[...]


def _copy_head(base, src):
    blk = pl.BlockSpec((_HEAD_BLK, H_DIM), lambda i: (i, 0))
    return pl.pallas_call(
        _head_copy_body,
        grid=(_HEAD // _HEAD_BLK,),
        in_specs=[pl.BlockSpec(memory_space=pl.ANY), blk],
        out_specs=blk,
        out_shape=jax.ShapeDtypeStruct((NUM_ENTS, H_DIM), jnp.float32),
        input_output_aliases={0: 0},
    )(base, src)


def _tail_copy_body(src_hbm, out_hbm, rows_v, sem, sem_w):
    wid = _worker_id()
    base = _HEAD + wid * _TPW
    n = _TPW // _CHUNK

    def issue(k):
        return pltpu.async_copy(
            src_hbm.at[pl.ds(base + k * _CHUNK, _CHUNK)],
            rows_v.at[k % 3], sem)

    gathers = [issue(0), issue(1)]
    writeouts = []
    for k in range(n):
        gathers[k].wait()
        if k >= 1:
            writeouts[k - 1].wait()
        if k + 2 < n:
            gathers.append(issue(k + 2))
        writeouts.append(pltpu.async_copy(
            rows_v.at[k % 3],
            out_hbm.at[pl.ds(base + k * _CHUNK, _CHUNK)], sem_w))
    writeouts[n - 1].wait()


def _sc_tail_copy(src):
    f = pl.kernel(
        _tail_copy_body,
        out_type=jax.ShapeDtypeStruct((NUM_ENTS, H_DIM), jnp.float32),
        mesh=_vmesh(),
        compiler_params=pltpu.CompilerParams(needs_layout_passes=False),
        scratch_types=[
            pltpu.VMEM((3, _CHUNK, H_DIM), jnp.float32),
            pltpu.SemaphoreType.DMA,
            pltpu.SemaphoreType.DMA,
        ],
    )
    return f(src)


_vmesh_cache = []


def _vmesh():
    if not _vmesh_cache:
        _vmesh_cache.append(plsc.VectorSubcoreMesh(
            core_axis_name="c", subcore_axis_name="s",
            num_cores=_NC, num_subcores=_NS))
    return _vmesh_cache[0]


def _worker_id():
    return lax.axis_index("s") * _NC + lax.axis_index("c")


def _gather_winner_body(ids1_hbm, slow_hbm, fast_hbm,
                        slow_rows_hbm, fast_rows_hbm, winner_hbm,
                        ids_v, sem):
    wid = _worker_id()
    base = wid * _RPW

    for j in range(_NCHUNK):
        pltpu.sync_copy(ids1_hbm.at[pl.ds(base + j * _CHUNK, _CHUNK)],
                        ids_v.at[j])

    def gather_scope(rows3, sem_g, sem_w):
        plan = [(src, dst, j)
                for src, dst in ((slow_hbm, slow_rows_hbm),
                                 (fast_hbm, fast_rows_hbm))
                for j in range(_NCHUNK)]
        n = len(plan)

        def issue(k):
            src, _, j = plan[k]
            return pltpu.async_copy(src.at[ids_v.at[j]],
                                    rows3.at[k % 4], sem_g)

        gathers = [issue(0), issue(1), issue(2)]
        writeouts = []
        for k in range(n):
            gathers[k].wait()
            if k >= 1:
                writeouts[k - 1].wait()
            if k + 3 < n:
                gathers.append(issue(k + 3))
            _, dst, j = plan[k]
            writeouts.append(pltpu.async_copy(
                rows3.at[k % 4],
                dst.at[pl.ds(base + j * _CHUNK, _CHUNK)], sem_w))
        writeouts[n - 1].wait()

    pl.run_scoped(
        gather_scope,
        pltpu.VMEM((4, _CHUNK, H_DIM), jnp.float32),
        pltpu.SemaphoreType.DMA,
        pltpu.SemaphoreType.DMA,
    )

    @pl.when(wid == 0)
    def _():
        def scoped(table, idbuf):
            ngrp = BATCH // (16 * _UNROLL)
            iota = lax.iota(jnp.int32, 16)
            pltpu.sync_copy(ids1_hbm, idbuf)

            def pass1(g, c2):
                for u in range(_UNROLL):
                    off = g * (16 * _UNROLL) + u * 16
                    idx = idbuf[pl.ds(off, 16)]
                    plsc.store_scatter(table, [idx], off + iota)
                return c2
            lax.fori_loop(0, ngrp, pass1, 0)

            def pass2(g, c2):
                for u in range(_UNROLL):
                    off = g * (16 * _UNROLL) + u * 16
                    idx = idbuf[pl.ds(off, 16)]
                    win = plsc.load_gather(table, [idx])
                    idbuf[pl.ds(off, 16)] = win
                return c2
            lax.fori_loop(0, ngrp, pass2, 0)
            pltpu.sync_copy(idbuf, winner_hbm)

        pl.run_scoped(
            scoped,
            pltpu.VMEM((NUM_ENTS,), jnp.int32),
            pltpu.VMEM((BATCH,), jnp.int32),
        )


def _sc_gather_winner(ids1, slow, fast):
    f = pl.kernel(
        _gather_winner_body,
        out_type=[
            jax.ShapeDtypeStruct((BATCH, H_DIM), jnp.float32),
            jax.ShapeDtypeStruct((BATCH, H_DIM), jnp.float32),
            jax.ShapeDtypeStruct((BATCH,), jnp.int32),
        ],
        mesh=_vmesh(),
        compiler_params=pltpu.CompilerParams(needs_layout_passes=False),
        scratch_types=[
            pltpu.VMEM((_NCHUNK, _CHUNK), jnp.int32),
            pltpu.SemaphoreType.DMA,
        ],
    )
    return f(ids1, slow, fast)


def _scatter_body(ids1_hbm, win1_hbm, upd_hbm, out_ref,
                  ids_v, win_v, rows_v, sem, sem2):
    wid = _worker_id()
    base = wid * _RPW
    for j in range(_NCHUNK):
        pltpu.sync_copy(ids1_hbm.at[pl.ds(base + j * _CHUNK, _CHUNK)],
                        ids_v.at[j])
        pltpu.sync_copy(win1_hbm.at[pl.ds(base + j * _CHUNK, _CHUNK)],
                        win_v.at[j])
    gathers = [pltpu.async_copy(upd_hbm.at[win_v.at[j]],
                                rows_v.at[j], sem)
               for j in range(_NCHUNK)]
    scatters = []
    for j in range(_NCHUNK):
        gathers[j].wait()
        scatters.append(pltpu.async_copy(
            rows_v.at[j], out_ref.at[ids_v.at[j]], sem2))
    for d in scatters:
        d.wait()


def _sc_scatter(ids1, win1, upd, out_ref):
    f = pl.kernel(
        _scatter_body,
        out_type=(),
        mesh=_vmesh(),
        compiler_params=pltpu.CompilerParams(needs_layout_passes=False),
        scratch_types=[
            pltpu.VMEM((_NCHUNK, _CHUNK), jnp.int32),
            pltpu.VMEM((_NCHUNK, _CHUNK), jnp.int32),
            pltpu.VMEM((_NCHUNK, _CHUNK, H_DIM), jnp.float32),
            pltpu.SemaphoreType.DMA,
            pltpu.SemaphoreType.DMA,
        ],
    )
    f(ids1, win1, upd, out_ref)


def kernel(entity_state_slow, entity_state_fast, slow_threshold, slow_scale,
           h_hyp, context_tangent, c, entity_ids):
    ids = entity_ids.astype(jnp.int32)

    scal = jnp.stack([c, slow_threshold, slow_scale]).astype(jnp.float32)

    slow_rows, fast_rows, winner = _sc_gather_winner(
        ids, entity_state_slow, entity_state_fast)

    slow_tail = _sc_tail_copy(entity_state_slow)

    newfast_base = _copy_buffer(entity_state_fast)

    h_out, fast_new, slow_upd = _dense_compute(
        scal, h_hyp, context_tangent, slow_rows, fast_rows)

    newfast_ref = jax.new_ref(newfast_base)
    _sc_scatter(ids, winner, fast_new, newfast_ref)

    newslow_base = _copy_head(slow_tail, entity_state_slow)
    newslow_ref = jax.new_ref(newslow_base)
    _sc_scatter(ids, winner, slow_upd, newslow_ref)
    return (h_out, jax.freeze(newfast_ref), jax.freeze(newslow_ref))


# speedup vs baseline: 1.0546x; 1.0546x over previous
import jax
import jax.numpy as jnp
from jax import lax
from jax.experimental import pallas as pl
from jax.experimental.pallas import tpu as pltpu
from jax.experimental.pallas import tpu_sc as plsc

NUM_ENTS = 100000
H_DIM = 128
BATCH = 16384
ALPHA = 0.2

_NC = 2
_NS = 16
_NW = _NC * _NS
_RPW = BATCH // _NW
_CHUNK = 128
_NCHUNK = _RPW // _CHUNK
_NROW = BATCH // _CHUNK

_WIN_CHUNK = 2048
_UNROLL = 8

_DENSE_BLK = 4096
_COPY_BLK = 10000


def _dense_body(scal_ref, h_ref, ctx_ref, slow_ref, fast_ref,
                hout_ref, fastnew_ref, slowupd_ref):
    sqrt_c = jnp.sqrt(scal_ref[0])
    thr = jnp.maximum(scal_ref[1], 1e-6)
    scale = jnp.maximum(scal_ref[2], 0.1)

    h = h_ref[...]
    slow = slow_ref[...]

    n1 = jnp.maximum(jnp.sqrt(jnp.sum(h * h, axis=-1, keepdims=True)), 1e-10)
    x = jnp.clip(sqrt_c * n1, -1.0 + 1e-7, 1.0 - 1e-7)
    artanh = 0.5 * jnp.log((1.0 + x) / (1.0 - x))
    t = artanh * h / (sqrt_c * n1) + slow
    t = jnp.clip(t, -10.0, 10.0)

    n2 = jnp.maximum(jnp.sqrt(jnp.sum(t * t, axis=-1, keepdims=True)), 1e-10)
    arg = jnp.clip(sqrt_c * n2, -15.0, 15.0)
    e = jnp.tanh(arg) * t / (sqrt_c * n2)

    n3 = jnp.maximum(jnp.sqrt(jnp.sum(e * e, axis=-1, keepdims=True)), 1e-10)
    max_norm = (1.0 - 1e-5) / sqrt_c
    hout_ref[...] = e * jnp.minimum(1.0, max_norm / n3)

    fast_new = (1.0 - ALPHA) * fast_ref[...] + ALPHA * ctx_ref[...]
    fastnew_ref[...] = fast_new
    delta = fast_new - slow
    dn = jnp.sqrt(jnp.sum(delta * delta, axis=-1, keepdims=True))
    gate = jax.nn.sigmoid(scale * (dn - thr))
    slowupd_ref[...] = slow + gate * delta


def _dense_compute(scal, h_hyp, context_tangent, slow_rows, fast_rows):
    grid = (BATCH // _DENSE_BLK,)
    row_spec = pl.BlockSpec((_DENSE_BLK, H_DIM), lambda i: (i, 0))
    return pl.pallas_call(
        _dense_body,
        grid=grid,
        in_specs=[
            pl.BlockSpec(memory_space=pltpu.SMEM),
            row_spec, row_spec, row_spec, row_spec,
        ],
        out_specs=[row_spec, row_spec, row_spec],
        out_shape=[
            jax.ShapeDtypeStruct((BATCH, H_DIM), jnp.float32),
            jax.ShapeDtypeStruct((BATCH, H_DIM), jnp.float32),
            jax.ShapeDtypeStruct((BATCH, H_DIM), jnp.float32),
        ],
    )(scal, h_hyp, context_tangent, slow_rows, fast_rows)


def _copy_body(src_ref, dst_ref):
    dst_ref[...] = src_ref[...]


def _copy_buffer(buf):
    blk = pl.BlockSpec((_COPY_BLK, H_DIM), lambda i: (i, 0))
    return pl.pallas_call(
        _copy_body,
        grid=(NUM_ENTS // _COPY_BLK,),
        in_specs=[blk],
        out_specs=blk,
        out_shape=jax.ShapeDtypeStruct((NUM_ENTS, H_DIM), jnp.float32),
    )(buf)


_TAIL = 49152
_HEAD = NUM_ENTS - _TAIL
_TPW = _TAIL // _NW
_HEAD_BLK = 7264


def _head_copy_body(_base_ref, src_ref, dst_ref):
    dst_ref[...] = src_ref[...]


def _copy_head(base, src):
    blk = pl.BlockSpec((_HEAD_BLK, H_DIM), lambda i: (i, 0))
    return pl.pallas_call(
        _head_copy_body,
        grid=(_HEAD // _HEAD_BLK,),
        in_specs=[pl.BlockSpec(memory_space=pl.ANY), blk],
        out_specs=blk,
        out_shape=jax.ShapeDtypeStruct((NUM_ENTS, H_DIM), jnp.float32),
        input_output_aliases={0: 0},
    )(base, src)


def _tail_copy_body(src_hbm, out_hbm, rows_v, sem, sem_w):
    wid = _worker_id()
    base = _HEAD + wid * _TPW
    n = _TPW // _CHUNK

    def issue(k):
        return pltpu.async_copy(
            src_hbm.at[pl.ds(base + k * _CHUNK, _CHUNK)],
            rows_v.at[k % 3], sem)

    gathers = [issue(0), issue(1)]
    writeouts = []
    for k in range(n):
        gathers[k].wait()
        if k >= 1:
            writeouts[k - 1].wait()
        if k + 2 < n:
            gathers.append(issue(k + 2))
        writeouts.append(pltpu.async_copy(
            rows_v.at[k % 3],
            out_hbm.at[pl.ds(base + k * _CHUNK, _CHUNK)], sem_w))
    writeouts[n - 1].wait()


def _sc_tail_copy(src):
    f = pl.kernel(
        _tail_copy_body,
        out_type=jax.ShapeDtypeStruct((NUM_ENTS, H_DIM), jnp.float32),
        mesh=_vmesh(),
        compiler_params=pltpu.CompilerParams(needs_layout_passes=False),
        scratch_types=[
            pltpu.VMEM((3, _CHUNK, H_DIM), jnp.float32),
            pltpu.SemaphoreType.DMA,
            pltpu.SemaphoreType.DMA,
        ],
    )
    return f(src)


_vmesh_cache = []


def _vmesh():
    if not _vmesh_cache:
        _vmesh_cache.append(plsc.VectorSubcoreMesh(
            core_axis_name="c", subcore_axis_name="s",
            num_cores=_NC, num_subcores=_NS))
    return _vmesh_cache[0]


def _worker_id():
    return lax.axis_index("s") * _NC + lax.axis_index("c")


def _gather_winner_body(ids1_hbm, slow_hbm, fast_hbm,
                        slow_rows_hbm, fast_rows_hbm, winner_hbm,
                        ids_v, sem):
    wid = _worker_id()
    base = wid * _RPW

    for j in range(_NCHUNK):
        pltpu.sync_copy(ids1_hbm.at[pl.ds(base + j * _CHUNK, _CHUNK)],
                        ids_v.at[j])

    def gather_scope(rows3, sem_g, sem_w):
        plan = [(src, dst, j)
                for src, dst in ((slow_hbm, slow_rows_hbm),
                                 (fast_hbm, fast_rows_hbm))
                for j in range(_NCHUNK)]
        n = len(plan)

        def issue(k):
            src, _, j = plan[k]
            return pltpu.async_copy(src.at[ids_v.at[j]],
                                    rows3.at[k % 4], sem_g)

        gathers = [issue(0), issue(1), issue(2)]
        writeouts = []
        for k in range(n):
            gathers[k].wait()
            if k >= 1:
                writeouts[k - 1].wait()
            if k + 3 < n:
                gathers.append(issue(k + 3))
            _, dst, j = plan[k]
            writeouts.append(pltpu.async_copy(
                rows3.at[k % 4],
                dst.at[pl.ds(base + j * _CHUNK, _CHUNK)], sem_w))
        writeouts[n - 1].wait()

    pl.run_scoped(
        gather_scope,
        pltpu.VMEM((4, _CHUNK, H_DIM), jnp.float32),
        pltpu.SemaphoreType.DMA,
        pltpu.SemaphoreType.DMA,
    )

    @pl.when(wid == 0)
    def _():
        def scoped(table, idbuf):
            ngrp = BATCH // (16 * _UNROLL)
            iota = lax.iota(jnp.int32, 16)
            pltpu.sync_copy(ids1_hbm, idbuf)

            def pass1(g, c2):
                for u in range(_UNROLL):
                    off = g * (16 * _UNROLL) + u * 16
                    idx = idbuf[pl.ds(off, 16)]
                    plsc.store_scatter(table, [idx], off + iota)
                return c2
            lax.fori_loop(0, ngrp, pass1, 0)

            def pass2(g, c2):
                for u in range(_UNROLL):
                    off = g * (16 * _UNROLL) + u * 16
                    idx = idbuf[pl.ds(off, 16)]
                    win = plsc.load_gather(table, [idx])
                    idbuf[pl.ds(off, 16)] = win
                return c2
            lax.fori_loop(0, ngrp, pass2, 0)
            pltpu.sync_copy(idbuf, winner_hbm)

        pl.run_scoped(
            scoped,
            pltpu.VMEM((NUM_ENTS,), jnp.int32),
            pltpu.VMEM((BATCH,), jnp.int32),
        )


def _sc_gather_winner(ids1, slow, fast):
    f = pl.kernel(
        _gather_winner_body,
        out_type=[
            jax.ShapeDtypeStruct((BATCH, H_DIM), jnp.float32),
            jax.ShapeDtypeStruct((BATCH, H_DIM), jnp.float32),
            jax.ShapeDtypeStruct((BATCH,), jnp.int32),
        ],
        mesh=_vmesh(),
        compiler_params=pltpu.CompilerParams(needs_layout_passes=False),
        scratch_types=[
            pltpu.VMEM((_NCHUNK, _CHUNK), jnp.int32),
            pltpu.SemaphoreType.DMA,
        ],
    )
    return f(ids1, slow, fast)


def _scatter_body(ids1_hbm, win1_hbm, upd_hbm, out_ref,
                  ids_v, win_v, rows_v, sem, sem2):
    wid = _worker_id()
    base = wid * _RPW
    for j in range(_NCHUNK):
        pltpu.sync_copy(ids1_hbm.at[pl.ds(base + j * _CHUNK, _CHUNK)],
                        ids_v.at[j])
        pltpu.sync_copy(win1_hbm.at[pl.ds(base + j * _CHUNK, _CHUNK)],
                        win_v.at[j])
    gathers = [pltpu.async_copy(upd_hbm.at[win_v.at[j]],
                                rows_v.at[j], sem)
               for j in range(_NCHUNK)]
    scatters = []
    for j in range(_NCHUNK):
        gathers[j].wait()
        scatters.append(pltpu.async_copy(
            rows_v.at[j], out_ref.at[ids_v.at[j]], sem2))
    for d in scatters:
        d.wait()


def _sc_scatter(ids1, win1, upd, out_ref):
    f = pl.kernel(
        _scatter_body,
        out_type=(),
        mesh=_vmesh(),
        compiler_params=pltpu.CompilerParams(needs_layout_passes=False),
        scratch_types=[
            pltpu.VMEM((_NCHUNK, _CHUNK), jnp.int32),
            pltpu.VMEM((_NCHUNK, _CHUNK), jnp.int32),
            pltpu.VMEM((_NCHUNK, _CHUNK, H_DIM), jnp.float32),
            pltpu.SemaphoreType.DMA,
            pltpu.SemaphoreType.DMA,
        ],
    )
    f(ids1, win1, upd, out_ref)


def kernel(entity_state_slow, entity_state_fast, slow_threshold, slow_scale,
           h_hyp, context_tangent, c, entity_ids):
    ids = entity_ids.astype(jnp.int32)

    scal = jnp.stack([c, slow_threshold, slow_scale]).astype(jnp.float32)

    slow_rows, fast_rows, winner = _sc_gather_winner(
        ids, entity_state_slow, entity_state_fast)

    slow_tail = _sc_tail_copy(entity_state_slow)

    newfast_base = _copy_buffer(entity_state_fast)

    h_out, fast_new, slow_upd = _dense_compute(
        scal, h_hyp, context_tangent, slow_rows, fast_rows)

    newfast_ref = jax.new_ref(newfast_base)
    _sc_scatter(ids, winner, fast_new, newfast_ref)

    newslow_base = _copy_head(slow_tail, entity_state_slow)
    newslow_ref = jax.new_ref(newslow_base)
    _sc_scatter(ids, winner, slow_upd, newslow_ref)
    return (h_out, jax.freeze(newfast_ref), jax.freeze(newslow_ref))
